# baseline (speedup 1.0000x reference)
import functools

import jax
import jax.numpy as jnp
from jax import lax
from jax.experimental import pallas as pl
from jax.experimental.pallas import tpu as pltpu
from jax.experimental.pallas import tpu_sc as plsc

B = 16384
D = 64
V_PAD = 1024


def _sc_gather_body(nc, bpw,
                    emb_u_hbm, emb_s_hbm, bias_u_hbm, bias_s_hbm,
                    uid_hbm, sid_hbm,
                    xe_u_hbm, xe_s_hbm, xb_hbm,
                    idx_u, idx_s, rows_u, rows_s, bu_v, bs_v, bsum_v,
                    sem_u, sem_s):
    wid = lax.axis_index("s") * nc + lax.axis_index("c")
    base = wid * bpw
    pltpu.sync_copy(uid_hbm.at[pl.ds(base, bpw)], idx_u)
    pltpu.sync_copy(sid_hbm.at[pl.ds(base, bpw)], idx_s)
    cp_u = pltpu.async_copy(emb_u_hbm.at[idx_u], rows_u, sem_u)
    cp_s = pltpu.async_copy(emb_s_hbm.at[idx_s], rows_s, sem_s)
    pltpu.sync_copy(bias_u_hbm, bu_v)
    pltpu.sync_copy(bias_s_hbm, bs_v)
    for i in range(bpw // 16):
        iu = idx_u[pl.ds(i * 16, 16)]
        isk = idx_s[pl.ds(i * 16, 16)]
        vb = plsc.load_gather(bu_v, [iu]) + plsc.load_gather(bs_v, [isk])
        bsum_v[pl.ds(i * 16, 16)] = vb
    pltpu.sync_copy(bsum_v, xb_hbm.at[pl.ds(base, bpw)])
    cp_u.wait()
    pltpu.sync_copy(rows_u, xe_u_hbm.at[pl.ds(base, bpw)])
    cp_s.wait()
    pltpu.sync_copy(rows_s, xe_s_hbm.at[pl.ds(base, bpw)])


@functools.cache
def _make_sc_gather():
    info = plsc.get_sparse_core_info()
    nc, ns = info.num_cores, info.num_subcores
    nw = nc * ns
    bpw = B // nw
    mesh = plsc.VectorSubcoreMesh(core_axis_name="c", subcore_axis_name="s",
                                  num_cores=nc)
    return pl.kernel(
        functools.partial(_sc_gather_body, nc, bpw),
        out_type=(
            jax.ShapeDtypeStruct((B, D), jnp.bfloat16),
            jax.ShapeDtypeStruct((B, D), jnp.bfloat16),
            jax.ShapeDtypeStruct((B,), jnp.float32),
        ),
        mesh=mesh,
        scratch_types=[
            pltpu.VMEM((bpw,), jnp.int32),
            pltpu.VMEM((bpw,), jnp.int32),
            pltpu.VMEM((bpw, D), jnp.bfloat16),
            pltpu.VMEM((bpw, D), jnp.bfloat16),
            pltpu.VMEM((V_PAD,), jnp.float32),
            pltpu.VMEM((V_PAD,), jnp.float32),
            pltpu.VMEM((bpw,), jnp.float32),
            pltpu.SemaphoreType.DMA,
            pltpu.SemaphoreType.DMA,
        ],
        compiler_params=pltpu.CompilerParams(use_tc_tiling_on_sc=False,
                                             needs_layout_passes=False,
                                             skip_device_barrier=True),
        name="sc_embed_gather",
    )


BLK = 8192


def _tc_body(uf_ref, sf_ref, wu_ref, bu_ref, ws_ref, bs_ref,
             xeu_ref, xes_ref, xb_ref, ones_ref, out_ref):
    xfu = jnp.maximum(
        jnp.dot(uf_ref[...], wu_ref[...],
                preferred_element_type=jnp.float32) + bu_ref[...], 0.0)
    xfs = jnp.maximum(
        jnp.dot(sf_ref[...], ws_ref[...],
                preferred_element_type=jnp.float32) + bs_ref[...], 0.0)
    eu = xeu_ref[...].astype(jnp.float32) + xfu
    es = xes_ref[...].astype(jnp.float32) + xfs
    prod = eu * es
    comb = jnp.dot(prod, ones_ref[...],
                   preferred_element_type=jnp.float32)[:, 0]
    out_ref[...] = 4.0 * jax.nn.sigmoid(xb_ref[...] + comb) + 1.0


def _tc_dense(uf, sf, wu, bu, ws, bs, xeu, xes, xb):
    nblk = B // BLK
    row64 = pl.BlockSpec((BLK, D), lambda i: (i, 0))
    ones = jnp.ones((D, 1), jnp.float32)
    return pl.pallas_call(
        _tc_body,
        grid=(nblk,),
        in_specs=[
            pl.BlockSpec((BLK, uf.shape[1]), lambda i: (i, 0)),
            pl.BlockSpec((BLK, sf.shape[1]), lambda i: (i, 0)),
            pl.BlockSpec(wu.shape, lambda i: (0, 0)),
            pl.BlockSpec(bu.shape, lambda i: (0, 0)),
            pl.BlockSpec(ws.shape, lambda i: (0, 0)),
            pl.BlockSpec(bs.shape, lambda i: (0, 0)),
            row64,
            row64,
            pl.BlockSpec((BLK,), lambda i: (i,)),
            pl.BlockSpec((D, 1), lambda i: (0, 0)),
        ],
        out_specs=pl.BlockSpec((BLK,), lambda i: (i,)),
        out_shape=jax.ShapeDtypeStruct((B,), jnp.float32),
        compiler_params=pltpu.CompilerParams(
            dimension_semantics=("arbitrary",),
        ),
        name="tc_rating_dense",
    )(uf, sf, wu, bu, ws, bs, xeu, xes, xb, ones)


def kernel(user_id, sku_id, user_features, sku_features, emb_user, emb_sku,
           bias_user, bias_sku, W_user, b_user, W_sku, b_sku):
    uid = user_id[:, 0].astype(jnp.int32)
    sid = sku_id[:, 0].astype(jnp.int32)
    bu_pad = jnp.pad(bias_user[:, 0], (0, V_PAD - bias_user.shape[0]))
    bs_pad = jnp.pad(bias_sku[:, 0], (0, V_PAD - bias_sku.shape[0]))
    xe_u, xe_s, xb = _make_sc_gather()(emb_user.astype(jnp.bfloat16),
                                       emb_sku.astype(jnp.bfloat16),
                                       bu_pad, bs_pad, uid, sid)
    return _tc_dense(user_features, sku_features,
                     W_user, b_user.reshape(1, D),
                     W_sku, b_sku.reshape(1, D),
                     xe_u, xe_s, xb)


# speedup vs baseline: 1.2032x; 1.2032x over previous
import functools

import jax
import jax.numpy as jnp
from jax import lax
from jax.experimental import pallas as pl
from jax.experimental.pallas import tpu as pltpu
from jax.experimental.pallas import tpu_sc as plsc

B = 16384
D = 64
V_PAD = 1024


def _sc_gather_body(nc, bpw,
                    emb_u_hbm, emb_s_hbm, bias_u_hbm, bias_s_hbm,
                    uid_hbm, sid_hbm,
                    xe_hbm, xb_hbm,
                    idx_u, idx_s, rows_u, rows_s, bu_v, bs_v, bsum_v,
                    sem_u, sem_s):
    wid = lax.axis_index("s") * nc + lax.axis_index("c")
    base = wid * bpw
    pltpu.sync_copy(uid_hbm.at[pl.ds(base, bpw)], idx_u)
    pltpu.sync_copy(sid_hbm.at[pl.ds(base, bpw)], idx_s)
    cp_u = pltpu.async_copy(emb_u_hbm.at[idx_u], rows_u, sem_u)
    cp_s = pltpu.async_copy(emb_s_hbm.at[idx_s], rows_s, sem_s)
    pltpu.sync_copy(bias_u_hbm, bu_v)
    pltpu.sync_copy(bias_s_hbm, bs_v)
    for i in range(bpw // 16):
        iu = idx_u[pl.ds(i * 16, 16)]
        isk = idx_s[pl.ds(i * 16, 16)]
        vb = plsc.load_gather(bu_v, [iu]) + plsc.load_gather(bs_v, [isk])
        bsum_v[pl.ds(i * 16, 16)] = vb
    pltpu.sync_copy(bsum_v, xb_hbm.at[pl.ds(base, bpw)])
    cp_u.wait()
    pltpu.sync_copy(rows_u, xe_hbm.at[pl.ds(base, bpw), pl.ds(0, D)])
    cp_s.wait()
    pltpu.sync_copy(rows_s, xe_hbm.at[pl.ds(base, bpw), pl.ds(D, D)])


@functools.cache
def _make_sc_gather():
    info = plsc.get_sparse_core_info()
    nc, ns = info.num_cores, info.num_subcores
    nw = nc * ns
    bpw = B // nw
    mesh = plsc.VectorSubcoreMesh(core_axis_name="c", subcore_axis_name="s",
                                  num_cores=nc)
    return pl.kernel(
        functools.partial(_sc_gather_body, nc, bpw),
        out_type=(
            jax.ShapeDtypeStruct((B, 2 * D), jnp.float32),
            jax.ShapeDtypeStruct((B,), jnp.float32),
        ),
        mesh=mesh,
        scratch_types=[
            pltpu.VMEM((bpw,), jnp.int32),
            pltpu.VMEM((bpw,), jnp.int32),
            pltpu.VMEM((bpw, D), jnp.float32),
            pltpu.VMEM((bpw, D), jnp.float32),
            pltpu.VMEM((V_PAD,), jnp.float32),
            pltpu.VMEM((V_PAD,), jnp.float32),
            pltpu.VMEM((bpw,), jnp.float32),
            pltpu.SemaphoreType.DMA,
            pltpu.SemaphoreType.DMA,
        ],
        compiler_params=pltpu.CompilerParams(use_tc_tiling_on_sc=False,
                                             needs_layout_passes=False,
                                             skip_device_barrier=True),
        name="sc_embed_gather",
    )


BLK = 4096


def _tc_body(uf_ref, sf_ref, wu_ref, bu_ref, ws_ref, bs_ref,
             xe_ref, xb_ref, ones_ref, out_ref):
    xfu = jnp.maximum(
        jnp.dot(uf_ref[...], wu_ref[...],
                preferred_element_type=jnp.float32) + bu_ref[...], 0.0)
    xfs = jnp.maximum(
        jnp.dot(sf_ref[...], ws_ref[...],
                preferred_element_type=jnp.float32) + bs_ref[...], 0.0)
    eu = xe_ref[:, :D] + xfu
    es = xe_ref[:, D:] + xfs
    prod = eu * es
    comb = jnp.dot(prod, ones_ref[...],
                   preferred_element_type=jnp.float32)[:, 0]
    out_ref[...] = 4.0 * jax.nn.sigmoid(xb_ref[...] + comb) + 1.0


def _tc_dense(uf, sf, wu, bu, ws, bs, xe, xb):
    nblk = B // BLK
    ones = jnp.ones((D, 1), jnp.float32)
    return pl.pallas_call(
        _tc_body,
        grid=(nblk,),
        in_specs=[
            pl.BlockSpec((BLK, uf.shape[1]), lambda i: (i, 0)),
            pl.BlockSpec((BLK, sf.shape[1]), lambda i: (i, 0)),
            pl.BlockSpec(wu.shape, lambda i: (0, 0)),
            pl.BlockSpec(bu.shape, lambda i: (0, 0)),
            pl.BlockSpec(ws.shape, lambda i: (0, 0)),
            pl.BlockSpec(bs.shape, lambda i: (0, 0)),
            pl.BlockSpec((BLK, 2 * D), lambda i: (i, 0)),
            pl.BlockSpec((BLK,), lambda i: (i,)),
            pl.BlockSpec((D, 1), lambda i: (0, 0)),
        ],
        out_specs=pl.BlockSpec((BLK,), lambda i: (i,)),
        out_shape=jax.ShapeDtypeStruct((B,), jnp.float32),
        compiler_params=pltpu.CompilerParams(
            dimension_semantics=("arbitrary",),
        ),
        name="tc_rating_dense",
    )(uf, sf, wu, bu, ws, bs, xe, xb, ones)


def kernel(user_id, sku_id, user_features, sku_features, emb_user, emb_sku,
           bias_user, bias_sku, W_user, b_user, W_sku, b_sku):
    uid = user_id[:, 0].astype(jnp.int32)
    sid = sku_id[:, 0].astype(jnp.int32)
    bu_pad = jnp.pad(bias_user[:, 0], (0, V_PAD - bias_user.shape[0]))
    bs_pad = jnp.pad(bias_sku[:, 0], (0, V_PAD - bias_sku.shape[0]))
    xe, xb = _make_sc_gather()(emb_user, emb_sku, bu_pad, bs_pad, uid, sid)
    return _tc_dense(user_features, sku_features,
                     W_user, b_user.reshape(1, D),
                     W_sku, b_sku.reshape(1, D),
                     xe, xb)
